# baseline (speedup 1.0000x reference)
import functools

import jax
import jax.numpy as jnp
from jax import lax
from jax.experimental import pallas as pl
from jax.experimental.pallas import tpu as pltpu
from jax.experimental.pallas import tpu_sc as plsc

NC, NS = 2, 16
NW = NC * NS
CH = 128
BN = 1024
NP = 10240
FRAC_FAST = 0.71


def _sc_mesh():
    return plsc.VectorSubcoreMesh(core_axis_name="c", subcore_axis_name="s",
                                  num_cores=NC, num_subcores=NS)


def _sc_degree(dst3, n, nch_by_core):
    nchunk = dst3.shape[1]
    rps = n // NS

    def body(dst_hbm, out_hbm, didx, ones_v, buf_v, acc_sh):
        c = lax.axis_index("c")
        s = lax.axis_index("s")
        w = c * NS + s
        nch = jnp.where(c == 0, nch_by_core[0], nch_by_core[1])

        def fill_o(i, carry):
            ones_v[pl.ds(i * 16, 16)] = jnp.ones((16,), jnp.float32)
            return carry
        lax.fori_loop(0, CH // 16, fill_o, 0)

        def fill_z(i, carry):
            buf_v[pl.ds(i * 16, 16)] = jnp.zeros((16,), jnp.float32)
            return carry
        lax.fori_loop(0, rps // 16, fill_z, 0)

        pltpu.sync_copy(dst_hbm.at[w], didx)
        pltpu.sync_copy(buf_v, acc_sh.at[pl.ds(s * rps, rps)])
        plsc.subcore_barrier()

        def step(j, carry):
            pltpu.sync_copy(ones_v, acc_sh.at[didx.at[j]], add=True)
            return carry
        lax.fori_loop(0, nch, step, 0)

        plsc.subcore_barrier()
        pltpu.sync_copy(acc_sh.at[pl.ds(s * rps, rps)], buf_v)
        pltpu.sync_copy(buf_v, out_hbm.at[pl.ds(c * n + s * rps, rps)])

    out = pl.kernel(
        body,
        out_type=jax.ShapeDtypeStruct((NC * n,), jnp.float32),
        mesh=_sc_mesh(),
        scratch_types=[
            pltpu.VMEM((nchunk, CH), jnp.int32),
            pltpu.VMEM((CH,), jnp.float32),
            pltpu.VMEM((rps,), jnp.float32),
            pltpu.VMEM_SHARED((n,), jnp.float32),
        ],
    )(dst3)
    return out.reshape(NC, n)


def _sc_propagate(y, pidx3, nch_by_core):
    n, d = y.shape
    nchunk = pidx3.shape[1]
    rps = n // NS
    nz = rps // CH

    def body(y_hbm, pidx_hbm, out_hbm, pidx, cb, rows0, rows1, acc_sh,
             sem0, sem1, sem2, sem3):
        c = lax.axis_index("c")
        s = lax.axis_index("s")
        w = c * NS + s
        npair = jnp.where(c == 0, (nch_by_core[0] - 1) // 2,
                          (nch_by_core[1] - 1) // 2)

        def fill_z(i, carry):
            for cc in range(d // 16):
                rows0[i, pl.ds(cc * 16, 16)] = jnp.zeros((16,), jnp.float32)
            return carry
        lax.fori_loop(0, CH, fill_z, 0)

        zcs = [pltpu.async_copy(rows0, acc_sh.at[pl.ds(s * rps + k * CH, CH)],
                                sem1) for k in range(nz)]
        pltpu.sync_copy(pidx_hbm.at[w], pidx)
        for z in zcs:
            z.wait()
        plsc.subcore_barrier()

        def unpack(j, sr, dr):
            for k in range(CH // 16):
                v = pidx[j, pl.ds(k * 16, 16)]
                cb[sr, pl.ds(k * 16, 16)] = lax.shift_right_logical(v, 14)
                cb[dr, pl.ds(k * 16, 16)] = lax.bitwise_and(v, (1 << 14) - 1)

        unpack(0, 0, 1)
        pltpu.async_copy(y_hbm.at[cb.at[0]], rows0, sem0)

        def step(p, carry):
            c1 = 2 * p + 1
            unpack(c1, 2, 3)
            pltpu.async_copy(y_hbm.at[cb.at[2]], rows1, sem1)
            pltpu.make_async_copy(y_hbm.at[cb.at[0]], rows0, sem0).wait()
            pltpu.sync_copy(rows0, acc_sh.at[cb.at[1]], add=True)
            unpack(c1 + 1, 0, 1)
            pltpu.async_copy(y_hbm.at[cb.at[0]], rows0, sem0)
            pltpu.make_async_copy(y_hbm.at[cb.at[2]], rows1, sem1).wait()
            pltpu.sync_copy(rows1, acc_sh.at[cb.at[3]], add=True)
            return carry
        lax.fori_loop(0, npair, step, 0)

        pltpu.make_async_copy(y_hbm.at[cb.at[0]], rows0, sem0).wait()
        pltpu.sync_copy(rows0, acc_sh.at[cb.at[1]], add=True)

        plsc.subcore_barrier()
        rbuf = [rows0, rows1]
        sin = [sem0, sem1]
        sout = [sem2, sem3]

        def acc_slice(k):
            return acc_sh.at[pl.ds(s * rps + k * CH, CH)]

        def hbm_slice(k):
            return out_hbm.at[pl.ds(c * n + s * rps + k * CH, CH)]

        ins = {}
        outs = {}
        ins[0] = pltpu.async_copy(acc_slice(0), rbuf[0], sin[0])
        if nz > 1:
            ins[1] = pltpu.async_copy(acc_slice(1), rbuf[1], sin[1])
        for k in range(nz):
            ins[k].wait()
            outs[k] = pltpu.async_copy(rbuf[k % 2], hbm_slice(k), sout[k % 2])
            if k + 2 < nz:
                outs[k].wait()
                ins[k + 2] = pltpu.async_copy(acc_slice(k + 2), rbuf[k % 2],
                                              sin[k % 2])
        for k in range(max(nz - 2, 0), nz):
            outs[k].wait()

    out = pl.kernel(
        body,
        out_type=jax.ShapeDtypeStruct((NC * n, d), jnp.float32),
        mesh=_sc_mesh(),
        scratch_types=[
            pltpu.VMEM((nchunk, CH), jnp.int32),
            pltpu.VMEM((4, CH), jnp.int32),
            pltpu.VMEM((CH, d), jnp.float32),
            pltpu.VMEM((CH, d), jnp.float32),
            pltpu.VMEM_SHARED((n, d), jnp.float32),
            pltpu.SemaphoreType.DMA,
            pltpu.SemaphoreType.DMA,
            pltpu.SemaphoreType.DMA,
            pltpu.SemaphoreType.DMA,
        ],
    )(y, pidx3)
    return out.reshape(NC, n, d)


def _tc_prep(degt, x):
    n, d = x.shape

    def body(degt_ref, x_ref, y_ref, dinv_ref):
        deg = degt_ref[:, 0:1] + degt_ref[:, 1:2] + 1.0
        dinv = lax.rsqrt(deg)
        dinv_ref[...] = dinv
        y_ref[...] = x_ref[...] * dinv

    grid = (n // BN,)
    return pl.pallas_call(
        body,
        grid=grid,
        in_specs=[
            pl.BlockSpec((BN, NC), lambda i: (i, 0)),
            pl.BlockSpec((BN, d), lambda i: (i, 0)),
        ],
        out_specs=[
            pl.BlockSpec((BN, d), lambda i: (i, 0)),
            pl.BlockSpec((BN, 1), lambda i: (i, 0)),
        ],
        out_shape=[
            jax.ShapeDtypeStruct((n, d), jnp.float32),
            jax.ShapeDtypeStruct((n, 1), jnp.float32),
        ],
    )(degt, x)


def _tc_combine(p, yself, dinv, W, b):
    n, d = yself.shape

    def body(p_ref, y_ref, dinv_ref, w_ref, b_ref, out_ref):
        dv = dinv_ref[...]
        agg = (p_ref[0] + p_ref[1] + y_ref[...]) * dv
        h = jnp.dot(agg, w_ref[...], preferred_element_type=jnp.float32)
        h = jnp.maximum(h + b_ref[...], 0.0)
        out_ref[...] = h * dv

    grid = (n // BN,)
    return pl.pallas_call(
        body,
        grid=grid,
        in_specs=[
            pl.BlockSpec((NC, BN, d), lambda i: (0, i, 0)),
            pl.BlockSpec((BN, d), lambda i: (i, 0)),
            pl.BlockSpec((BN, 1), lambda i: (i, 0)),
            pl.BlockSpec((d, d), lambda i: (0, 0)),
            pl.BlockSpec((1, d), lambda i: (0, 0)),
        ],
        out_specs=pl.BlockSpec((BN, d), lambda i: (i, 0)),
        out_shape=jax.ShapeDtypeStruct((n, d), jnp.float32),
    )(p, yself, dinv, W, b)


def _tc_final(p, yself, dinv, W2, b2, W3, b3):
    n, d = yself.shape
    c_out = W3.shape[1]

    def body(p_ref, y_ref, dinv_ref, w2_ref, b2_ref, w3_ref, b3_ref, out_ref):
        dv = dinv_ref[...]
        agg = (p_ref[0] + p_ref[1] + y_ref[...]) * dv
        h = jnp.dot(agg, w2_ref[...], preferred_element_type=jnp.float32)
        h = jnp.maximum(h + b2_ref[...], 0.0)
        logits = jnp.dot(h, w3_ref[...], preferred_element_type=jnp.float32)
        logits = logits + b3_ref[...]
        m = jnp.max(logits, axis=1, keepdims=True)
        lse = m + jnp.log(jnp.sum(jnp.exp(logits - m), axis=1, keepdims=True))
        out_ref[...] = logits - lse

    grid = (n // BN,)
    return pl.pallas_call(
        body,
        grid=grid,
        in_specs=[
            pl.BlockSpec((NC, BN, d), lambda i: (0, i, 0)),
            pl.BlockSpec((BN, d), lambda i: (i, 0)),
            pl.BlockSpec((BN, 1), lambda i: (i, 0)),
            pl.BlockSpec((d, d), lambda i: (0, 0)),
            pl.BlockSpec((1, d), lambda i: (0, 0)),
            pl.BlockSpec((d, c_out), lambda i: (0, 0)),
            pl.BlockSpec((1, c_out), lambda i: (0, 0)),
        ],
        out_specs=pl.BlockSpec((BN, c_out), lambda i: (i, 0)),
        out_shape=jax.ShapeDtypeStruct((n, c_out), jnp.float32),
    )(p, yself, dinv, W2, b2, W3, b3)


def kernel(x, edge_index, W1, b1, W2, b2, W3, b3):
    n, d = x.shape
    e = edge_index.shape[1]
    src = edge_index[0]
    dst = edge_index[1]

    xp = jnp.pad(x, ((0, NP - n), (0, 0)))
    nchf = min(int(round(e * FRAC_FAST / (NS * CH))), -(-e // (NS * CH)))
    nchf += 1 - (nchf % 2)
    capf = NS * nchf * CH
    take = min(e, capf)
    nchs = max(-(-(e - take) // (NS * CH)), 1)
    nchs += 1 - (nchs % 2)
    caps = NS * nchs * CH
    nmax = max(nchf, nchs)
    pv = (n << 14) | n
    pk = (src << 14) | dst
    fastm = jnp.pad(pk[:take], (0, capf - take),
                    constant_values=pv).reshape(NS, nchf, CH)
    slowm = jnp.pad(pk[take:], (0, NS * nchs * CH - (e - take)),
                    constant_values=pv).reshape(NS, nchs, CH)
    fastm = jnp.pad(fastm, ((0, 0), (0, nmax - nchf), (0, 0)),
                    constant_values=pv)
    slowm = jnp.pad(slowm, ((0, 0), (0, nmax - nchs), (0, 0)),
                    constant_values=pv)
    pidx3 = jnp.concatenate([fastm, slowm], axis=0)
    dst3 = pidx3 & ((1 << 14) - 1)
    nbc = (nchf, nchs)
    degp = _sc_degree(dst3, NP, nbc)
    y1, dinv = _tc_prep(degp.T, xp)
    p1 = _sc_propagate(y1, pidx3, nbc)
    y2 = _tc_combine(p1, y1, dinv, W1, b1.reshape(1, d))
    p2 = _sc_propagate(y2, pidx3, nbc)
    out = _tc_final(p2, y2, dinv, W2, b2.reshape(1, d), W3, b3.reshape(1, -1))
    return out[:n]
